# baseline (speedup 1.0000x reference)
import functools

import jax
import jax.numpy as jnp
from jax import lax
from jax.experimental import pallas as pl
from jax.experimental.pallas import tpu as pltpu
from jax.experimental.pallas import tpu_sc as plsc

RADIUS = 0.1
EXTENT = 0.12
K_PTS = 15
NEIGH = 32
NEG = 0.1

_ROWS = 64
_NB = 256
_SUB = 8
_CH = 512
_NW = 32


def _lrelu(v):
    return jnp.where(v >= 0, v, NEG * v)


_KROWS = 128
_KG = 16
_KT = 11
_BIGI = 2**30


def _knn_body(call_ref, cr_ref, out_ref, d2s, giota, cv, ci, idxacc):
    b = pl.program_id(0)
    n = call_ref.shape[1]
    gsz = n // _KG
    base = b * n
    inf = jnp.float32(jnp.inf)

    rx = cr_ref[0, 0, :].reshape(1, _KROWS)
    ry = cr_ref[0, 1, :].reshape(1, _KROWS)
    rz = cr_ref[0, 2, :].reshape(1, _KROWS)
    cx = call_ref[0, :, 0:1]
    cy = call_ref[0, :, 1:2]
    cz = call_ref[0, :, 2:3]
    dx = cx - rx
    dy = cy - ry
    dz = cz - rz
    d2 = dx * dx + dy * dy + dz * dz
    d2s[...] = d2.reshape(_KG, gsz, _KROWS)
    giota[...] = (
        lax.broadcasted_iota(jnp.int32, (_KG, gsz, _KROWS), 0) * gsz
        + lax.broadcasted_iota(jnp.int32, (_KG, gsz, _KROWS), 1)
    )

    imod = lax.broadcasted_iota(jnp.int32, (_KG, _KT, _KROWS), 1)

    def p1_round(t, _):
        dd = d2s[...]
        m = jnp.min(dd, axis=1, keepdims=True)
        am = jnp.min(
            jnp.where(dd == m, giota[...], _BIGI), axis=1, keepdims=True
        )
        cv[...] = jnp.where(imod == t, jnp.broadcast_to(m, (_KG, _KT, _KROWS)), cv[...])
        ci[...] = jnp.where(imod == t, jnp.broadcast_to(am, (_KG, _KT, _KROWS)), ci[...])
        d2s[...] = jnp.where(giota[...] == am, inf, dd)
        return 0

    lax.fori_loop(0, _KT, p1_round, 0, unroll=False)

    vlast = cv[:, _KT - 1, :]
    ilast = ci[:, _KT - 1, :]

    iota32 = lax.broadcasted_iota(jnp.int32, (NEIGH, _KROWS), 0)

    def p2_round(k, carry):
        cvv = cv[...].reshape(_KG * _KT, _KROWS)
        cii = ci[...].reshape(_KG * _KT, _KROWS)
        m = jnp.min(cvv, axis=0, keepdims=True)
        amidx = jnp.min(jnp.where(cvv == m, cii, _BIGI), axis=0, keepdims=True)
        idxacc[...] = jnp.where(iota32 == k, amidx + base, idxacc[...])
        sel = (cvv == m) & (cii == amidx)
        cv[...] = jnp.where(sel, inf, cvv).reshape(_KG, _KT, _KROWS)
        return (m, amidx)

    v32, i32 = lax.fori_loop(
        0,
        NEIGH,
        p2_round,
        (jnp.zeros((1, _KROWS), jnp.float32), jnp.zeros((1, _KROWS), jnp.int32)),
        unroll=False,
    )

    danger = (vlast < v32) | ((vlast == v32) & (ilast <= i32))
    need_fallback = jnp.max(jnp.where(danger, 1, 0)) > 0

    @pl.when(need_fallback)
    def _fallback():
        fx = call_ref[0, :, 0:1] - rx
        fy = call_ref[0, :, 1:2] - ry
        fz = call_ref[0, :, 2:3] - rz
        d2s[...] = (fx * fx + fy * fy + fz * fz).reshape(_KG, gsz, _KROWS)

        def full_round(k, _):
            dd = d2s[...].reshape(n, _KROWS)
            gi = giota[...].reshape(n, _KROWS)
            m = jnp.min(dd, axis=0, keepdims=True)
            am = jnp.min(jnp.where(dd == m, gi, _BIGI), axis=0, keepdims=True)
            idxacc[...] = jnp.where(iota32 == k, am + base, idxacc[...])
            d2s[...] = jnp.where(gi == am, inf, dd).reshape(_KG, gsz, _KROWS)
            return 0

        lax.fori_loop(0, NEIGH, full_round, 0, unroll=False)

    out_ref[0] = idxacc[...]


def _knn(coords, coords_t):
    b, n, _ = coords.shape
    return pl.pallas_call(
        _knn_body,
        grid=(b, n // _KROWS),
        in_specs=[
            pl.BlockSpec((1, n, 3), lambda i, j: (i, 0, 0)),
            pl.BlockSpec((1, 3, _KROWS), lambda i, j: (i, 0, j)),
        ],
        out_specs=pl.BlockSpec((1, NEIGH, _KROWS), lambda i, j: (i, 0, j)),
        out_shape=jax.ShapeDtypeStruct((b, NEIGH, n), jnp.int32),
        scratch_shapes=[
            pltpu.VMEM((_KG, n // _KG, _KROWS), jnp.float32),
            pltpu.VMEM((_KG, n // _KG, _KROWS), jnp.int32),
            pltpu.VMEM((_KG, _KT, _KROWS), jnp.float32),
            pltpu.VMEM((_KG, _KT, _KROWS), jnp.int32),
            pltpu.VMEM((NEIGH, _KROWS), jnp.int32),
        ],
    )(coords, coords_t)


def _sc_gather(table, idx, tiled=True):
    m = idx.shape[0]
    d = table.shape[1]
    m_w = m // _NW
    mesh = plsc.VectorSubcoreMesh(core_axis_name="c", subcore_axis_name="s")

    @functools.partial(
        pl.kernel,
        out_type=jax.ShapeDtypeStruct((m, d), jnp.float32),
        mesh=mesh,
        scratch_types=[
            pltpu.VMEM((_CH,), jnp.int32),
            pltpu.VMEM((_CH, d), jnp.float32),
            pltpu.SemaphoreType.DMA,
        ],
        compiler_params=pltpu.CompilerParams(use_tc_tiling_on_sc=tiled),
    )
    def gather_k(table_hbm, idx_hbm, out_hbm, idx_v, rows_v, sem):
        wid = lax.axis_index("s") * 2 + lax.axis_index("c")
        base = wid * m_w

        def body(i, _):
            off = base + i * _CH
            pltpu.sync_copy(idx_hbm.at[pl.ds(off, _CH)], idx_v)
            pltpu.async_copy(table_hbm.at[idx_v], rows_v, sem).wait()
            pltpu.sync_copy(rows_v, out_hbm.at[pl.ds(off, _CH)])
            return 0

        lax.fori_loop(0, m_w // _CH, body, 0)

    return gather_k(table, idx)


def _wprep_body(nc_ref, pts_ref, kp_ref, out_ref):
    nb = nc_ref[...].reshape(_NB, NEIGH, 16)
    ox = nb[:, :, 0] - pts_ref[:, 0:1]
    oy = nb[:, :, 1] - pts_ref[:, 1:2]
    oz = nb[:, :, 2] - pts_ref[:, 2:3]
    for p in range(K_PTS):
        ex = ox - kp_ref[p : p + 1, 0:1]
        ey = oy - kp_ref[p : p + 1, 1:2]
        ez = oz - kp_ref[p : p + 1, 2:3]
        dd = ex * ex + ey * ey + ez * ez
        dist = jnp.sqrt(dd + 1e-12)
        out_ref[0, p] = jnp.maximum(0.0, 1.0 - dist / EXTENT)


def _wprep(neigh_c, pts_flat, kp, b, n):
    blocks = (b * n) // _NB
    return pl.pallas_call(
        _wprep_body,
        grid=(blocks,),
        in_specs=[
            pl.BlockSpec((_NB * NEIGH, 16), lambda i: (i, 0)),
            pl.BlockSpec((_NB, 3), lambda i: (i, 0)),
            pl.BlockSpec((K_PTS, 3), lambda i: (0, 0)),
        ],
        out_specs=pl.BlockSpec(
            (1, K_PTS, _NB, NEIGH),
            lambda i: (i // (n // _NB), 0, i % (n // _NB), 0),
        ),
        out_shape=jax.ShapeDtypeStruct((b, K_PTS, n, NEIGH), jnp.float32),
    )(neigh_c, pts_flat, kp)


def _dense_body(x_ref, w_ref, out_ref):
    h = jnp.dot(x_ref[...], w_ref[...], preferred_element_type=jnp.float32)
    out_ref[...] = _lrelu(h)


def _dense_down(x, wd):
    m, ic = x.shape
    oc = wd.shape[1]
    blk = 1024
    return pl.pallas_call(
        _dense_body,
        grid=(m // blk,),
        in_specs=[
            pl.BlockSpec((blk, ic), lambda i: (i, 0)),
            pl.BlockSpec((ic, oc), lambda i: (0, 0)),
        ],
        out_specs=pl.BlockSpec((blk, oc), lambda i: (i, 0)),
        out_shape=jax.ShapeDtypeStruct((m, oc), jnp.float32),
    )(x, wd)


def _conv_body(nx_ref, wt_ref, x_ref, wk_ref, wup_ref, wsc_ref, rep_ref,
               mask_ref, out_ref, w2):
    mid = nx_ref.shape[1]
    nsub = _NB // _SUB
    rep = rep_ref[...]
    bmask = mask_ref[...]
    for s in range(nsub):
        a = wt_ref[0, :, s * _SUB : (s + 1) * _SUB, :].reshape(K_PTS * _SUB, NEIGH)
        wtile = jnp.dot(a, rep, preferred_element_type=jnp.float32) * bmask
        nx8 = nx_ref[pl.ds(s * _SUB * NEIGH, _SUB * NEIGH), :]
        wtd = jnp.dot(wtile, nx8, preferred_element_type=jnp.float32)
        w2[:, s * _SUB : (s + 1) * _SUB, :] = wtd.reshape(K_PTS, _SUB, mid)
    conv = jnp.zeros((_NB, mid), jnp.float32)
    for p in range(K_PTS):
        conv = conv + jnp.dot(w2[p], wk_ref[p], preferred_element_type=jnp.float32)
    conv = _lrelu(conv)
    up = jnp.dot(conv, wup_ref[...], preferred_element_type=jnp.float32)
    skip = jnp.dot(x_ref[...], wsc_ref[...], preferred_element_type=jnp.float32)
    out_ref[...] = _lrelu(up + skip)


def _conv_block(nx, wt, x, wk, wup, wsc, rep, bmask, b, n):
    mid = nx.shape[1]
    oc = wup.shape[1]
    ic = x.shape[1]
    blocks = (b * n) // _NB
    npb = n // _NB
    return pl.pallas_call(
        _conv_body,
        grid=(blocks,),
        in_specs=[
            pl.BlockSpec((_NB * NEIGH, mid), lambda i: (i, 0)),
            pl.BlockSpec(
                (1, K_PTS, _NB, NEIGH),
                lambda i: (i // npb, 0, i % npb, 0),
            ),
            pl.BlockSpec((_NB, ic), lambda i: (i, 0)),
            pl.BlockSpec((K_PTS, mid, mid), lambda i: (0, 0, 0)),
            pl.BlockSpec((mid, oc), lambda i: (0, 0)),
            pl.BlockSpec((ic, oc), lambda i: (0, 0)),
            pl.BlockSpec((NEIGH, _SUB * NEIGH), lambda i: (0, 0)),
            pl.BlockSpec((K_PTS * _SUB, _SUB * NEIGH), lambda i: (0, 0)),
        ],
        out_specs=pl.BlockSpec((_NB, oc), lambda i: (i, 0)),
        out_shape=jax.ShapeDtypeStruct((b * n, oc), jnp.float32),
        scratch_shapes=[pltpu.VMEM((K_PTS, _NB, mid), jnp.float32)],
    )(nx, wt, x, wk, wup, wsc, rep, bmask)


def kernel(coords, features, KP, W_down1, Wk1, Wup1, Wsc1, W_down2, Wk2,
           Wup2, Wsc2, W_down3, Wk3, Wup3, Wsc3):
    b, n, _ = coords.shape
    m = b * n * NEIGH

    coords_t = jnp.transpose(coords, (0, 2, 1))
    gidx = _knn(coords, coords_t)
    flat_idx = jnp.transpose(gidx, (0, 2, 1)).reshape(m)

    pts_flat = coords.reshape(b * n, 3)
    pts_pad = jnp.concatenate(
        [pts_flat, jnp.zeros((b * n, 13), jnp.float32)], axis=1
    )
    neigh_c = _sc_gather(pts_pad, flat_idx, tiled=False)
    wt = _wprep(neigh_c, pts_flat, KP, b, n)

    rep = (
        lax.broadcasted_iota(jnp.int32, (NEIGH, _SUB * NEIGH), 1) % NEIGH
        == lax.broadcasted_iota(jnp.int32, (NEIGH, _SUB * NEIGH), 0)
    ).astype(jnp.float32)
    bmask = (
        lax.broadcasted_iota(jnp.int32, (K_PTS * _SUB, _SUB * NEIGH), 0) % _SUB
        == lax.broadcasted_iota(jnp.int32, (K_PTS * _SUB, _SUB * NEIGH), 1)
        // NEIGH
    ).astype(jnp.float32)

    x = features.reshape(b * n, features.shape[2])
    for wd, wk, wup, wsc in (
        (W_down1, Wk1, Wup1, Wsc1),
        (W_down2, Wk2, Wup2, Wsc2),
        (W_down3, Wk3, Wup3, Wsc3),
    ):
        h = _dense_down(x, wd)
        nx = _sc_gather(h, flat_idx)
        x = _conv_block(nx, wt, x, wk, wup, wsc, rep, bmask, b, n)

    return x.reshape(b, n, -1)


# speedup vs baseline: 1.1462x; 1.1462x over previous
import functools

import jax
import jax.numpy as jnp
from jax import lax
from jax.experimental import pallas as pl
from jax.experimental.pallas import tpu as pltpu
from jax.experimental.pallas import tpu_sc as plsc

RADIUS = 0.1
EXTENT = 0.12
K_PTS = 15
NEIGH = 32
NEG = 0.1

_ROWS = 64
_NB = 256
_SUB = 8
_CH = 512
_NW = 32


def _lrelu(v):
    return jnp.where(v >= 0, v, NEG * v)


_KROWS = 128
_KG = 16
_KT = 12
_BIGI = 2**30


def _knn_body(call_ref, cr_ref, out_ref, d2s, giota, cv, ci, idxacc):
    b = pl.program_id(0)
    n = call_ref.shape[1]
    gsz = n // _KG
    base = b * n
    inf = jnp.float32(jnp.inf)

    rx = cr_ref[0, 0, :].reshape(1, _KROWS)
    ry = cr_ref[0, 1, :].reshape(1, _KROWS)
    rz = cr_ref[0, 2, :].reshape(1, _KROWS)
    cx = call_ref[0, :, 0:1]
    cy = call_ref[0, :, 1:2]
    cz = call_ref[0, :, 2:3]
    dx = cx - rx
    dy = cy - ry
    dz = cz - rz
    d2 = dx * dx + dy * dy + dz * dz
    d2s[...] = d2.reshape(_KG, gsz, _KROWS)
    giota[...] = (
        lax.broadcasted_iota(jnp.int32, (_KG, gsz, _KROWS), 0) * gsz
        + lax.broadcasted_iota(jnp.int32, (_KG, gsz, _KROWS), 1)
    )

    imod = lax.broadcasted_iota(jnp.int32, (_KG, _KT, _KROWS), 1)

    def p1_round(t, _):
        dd = d2s[...]
        m = jnp.min(dd, axis=1, keepdims=True)
        am = jnp.min(
            jnp.where(dd == m, giota[...], _BIGI), axis=1, keepdims=True
        )
        cv[...] = jnp.where(imod == t, jnp.broadcast_to(m, (_KG, _KT, _KROWS)), cv[...])
        ci[...] = jnp.where(imod == t, jnp.broadcast_to(am, (_KG, _KT, _KROWS)), ci[...])
        d2s[...] = jnp.where(giota[...] == am, inf, dd)
        return 0

    lax.fori_loop(0, _KT, p1_round, 0, unroll=False)

    vlast = cv[:, _KT - 1, :]
    ilast = ci[:, _KT - 1, :]

    iota32 = lax.broadcasted_iota(jnp.int32, (NEIGH, _KROWS), 0)

    def p2_round(k, carry):
        cvv = cv[...].reshape(_KG * _KT, _KROWS)
        cii = ci[...].reshape(_KG * _KT, _KROWS)
        m = jnp.min(cvv, axis=0, keepdims=True)
        amidx = jnp.min(jnp.where(cvv == m, cii, _BIGI), axis=0, keepdims=True)
        idxacc[...] = jnp.where(iota32 == k, amidx + base, idxacc[...])
        sel = (cvv == m) & (cii == amidx)
        cv[...] = jnp.where(sel, inf, cvv).reshape(_KG, _KT, _KROWS)
        return (m, amidx)

    v32, i32 = lax.fori_loop(
        0,
        NEIGH,
        p2_round,
        (jnp.zeros((1, _KROWS), jnp.float32), jnp.zeros((1, _KROWS), jnp.int32)),
        unroll=False,
    )

    danger = (vlast < v32) | ((vlast == v32) & (ilast <= i32))
    need_fallback = jnp.max(jnp.where(danger, 1, 0)) > 0

    @pl.when(need_fallback)
    def _fallback():
        fx = call_ref[0, :, 0:1] - rx
        fy = call_ref[0, :, 1:2] - ry
        fz = call_ref[0, :, 2:3] - rz
        d2s[...] = (fx * fx + fy * fy + fz * fz).reshape(_KG, gsz, _KROWS)

        def full_round(k, _):
            dd = d2s[...].reshape(n, _KROWS)
            gi = giota[...].reshape(n, _KROWS)
            m = jnp.min(dd, axis=0, keepdims=True)
            am = jnp.min(jnp.where(dd == m, gi, _BIGI), axis=0, keepdims=True)
            idxacc[...] = jnp.where(iota32 == k, am + base, idxacc[...])
            d2s[...] = jnp.where(gi == am, inf, dd).reshape(_KG, gsz, _KROWS)
            return 0

        lax.fori_loop(0, NEIGH, full_round, 0, unroll=False)

    out_ref[0] = idxacc[...]


def _knn(coords, coords_t):
    b, n, _ = coords.shape
    return pl.pallas_call(
        _knn_body,
        grid=(b, n // _KROWS),
        in_specs=[
            pl.BlockSpec((1, n, 3), lambda i, j: (i, 0, 0)),
            pl.BlockSpec((1, 3, _KROWS), lambda i, j: (i, 0, j)),
        ],
        out_specs=pl.BlockSpec((1, NEIGH, _KROWS), lambda i, j: (i, 0, j)),
        out_shape=jax.ShapeDtypeStruct((b, NEIGH, n), jnp.int32),
        scratch_shapes=[
            pltpu.VMEM((_KG, n // _KG, _KROWS), jnp.float32),
            pltpu.VMEM((_KG, n // _KG, _KROWS), jnp.int32),
            pltpu.VMEM((_KG, _KT, _KROWS), jnp.float32),
            pltpu.VMEM((_KG, _KT, _KROWS), jnp.int32),
            pltpu.VMEM((NEIGH, _KROWS), jnp.int32),
        ],
    )(coords, coords_t)


def _sc_gather(table, idx, tiled=True):
    m = idx.shape[0]
    d = table.shape[1]
    m_w = m // _NW
    mesh = plsc.VectorSubcoreMesh(core_axis_name="c", subcore_axis_name="s")

    @functools.partial(
        pl.kernel,
        out_type=jax.ShapeDtypeStruct((m, d), jnp.float32),
        mesh=mesh,
        scratch_types=[
            pltpu.VMEM((_CH,), jnp.int32),
            pltpu.VMEM((_CH, d), jnp.float32),
            pltpu.SemaphoreType.DMA,
        ],
        compiler_params=pltpu.CompilerParams(use_tc_tiling_on_sc=tiled),
    )
    def gather_k(table_hbm, idx_hbm, out_hbm, idx_v, rows_v, sem):
        wid = lax.axis_index("s") * 2 + lax.axis_index("c")
        base = wid * m_w

        def body(i, _):
            off = base + i * _CH
            pltpu.sync_copy(idx_hbm.at[pl.ds(off, _CH)], idx_v)
            pltpu.async_copy(table_hbm.at[idx_v], rows_v, sem).wait()
            pltpu.sync_copy(rows_v, out_hbm.at[pl.ds(off, _CH)])
            return 0

        lax.fori_loop(0, m_w // _CH, body, 0)

    return gather_k(table, idx)


def _wprep_body(nc_ref, pts_ref, kp_ref, out_ref):
    nb = nc_ref[...].reshape(_NB, NEIGH, 16)
    ox = nb[:, :, 0] - pts_ref[:, 0:1]
    oy = nb[:, :, 1] - pts_ref[:, 1:2]
    oz = nb[:, :, 2] - pts_ref[:, 2:3]
    for p in range(K_PTS):
        ex = ox - kp_ref[p : p + 1, 0:1]
        ey = oy - kp_ref[p : p + 1, 1:2]
        ez = oz - kp_ref[p : p + 1, 2:3]
        dd = ex * ex + ey * ey + ez * ez
        dist = jnp.sqrt(dd + 1e-12)
        out_ref[0, p] = jnp.maximum(0.0, 1.0 - dist / EXTENT)


def _wprep(neigh_c, pts_flat, kp, b, n):
    blocks = (b * n) // _NB
    return pl.pallas_call(
        _wprep_body,
        grid=(blocks,),
        in_specs=[
            pl.BlockSpec((_NB * NEIGH, 16), lambda i: (i, 0)),
            pl.BlockSpec((_NB, 3), lambda i: (i, 0)),
            pl.BlockSpec((K_PTS, 3), lambda i: (0, 0)),
        ],
        out_specs=pl.BlockSpec(
            (1, K_PTS, _NB, NEIGH),
            lambda i: (i // (n // _NB), 0, i % (n // _NB), 0),
        ),
        out_shape=jax.ShapeDtypeStruct((b, K_PTS, n, NEIGH), jnp.float32),
    )(neigh_c, pts_flat, kp)


def _dense_body(x_ref, w_ref, out_ref):
    h = jnp.dot(x_ref[...], w_ref[...], preferred_element_type=jnp.float32)
    out_ref[...] = _lrelu(h)


def _dense_down(x, wd):
    m, ic = x.shape
    oc = wd.shape[1]
    blk = 1024
    return pl.pallas_call(
        _dense_body,
        grid=(m // blk,),
        in_specs=[
            pl.BlockSpec((blk, ic), lambda i: (i, 0)),
            pl.BlockSpec((ic, oc), lambda i: (0, 0)),
        ],
        out_specs=pl.BlockSpec((blk, oc), lambda i: (i, 0)),
        out_shape=jax.ShapeDtypeStruct((m, oc), jnp.float32),
    )(x, wd)


def _conv_body(nx_ref, wt_ref, x_ref, wk_ref, wup_ref, wsc_ref, rep_ref,
               mask_ref, out_ref, w2):
    mid = nx_ref.shape[1]
    nsub = _NB // _SUB
    rep = rep_ref[...]
    bmask = mask_ref[...]
    for s in range(nsub):
        a = wt_ref[0, :, s * _SUB : (s + 1) * _SUB, :].reshape(K_PTS * _SUB, NEIGH)
        wtile = jnp.dot(a, rep, preferred_element_type=jnp.float32) * bmask
        nx8 = nx_ref[pl.ds(s * _SUB * NEIGH, _SUB * NEIGH), :]
        wtd = jnp.dot(wtile, nx8, preferred_element_type=jnp.float32)
        w2[:, s * _SUB : (s + 1) * _SUB, :] = wtd.reshape(K_PTS, _SUB, mid)
    conv = jnp.zeros((_NB, mid), jnp.float32)
    for p in range(K_PTS):
        conv = conv + jnp.dot(w2[p], wk_ref[p], preferred_element_type=jnp.float32)
    conv = _lrelu(conv)
    up = jnp.dot(conv, wup_ref[...], preferred_element_type=jnp.float32)
    skip = jnp.dot(x_ref[...], wsc_ref[...], preferred_element_type=jnp.float32)
    out_ref[...] = _lrelu(up + skip)


def _conv_block(nx, wt, x, wk, wup, wsc, rep, bmask, b, n):
    mid = nx.shape[1]
    oc = wup.shape[1]
    ic = x.shape[1]
    blocks = (b * n) // _NB
    npb = n // _NB
    return pl.pallas_call(
        _conv_body,
        grid=(blocks,),
        in_specs=[
            pl.BlockSpec((_NB * NEIGH, mid), lambda i: (i, 0)),
            pl.BlockSpec(
                (1, K_PTS, _NB, NEIGH),
                lambda i: (i // npb, 0, i % npb, 0),
            ),
            pl.BlockSpec((_NB, ic), lambda i: (i, 0)),
            pl.BlockSpec((K_PTS, mid, mid), lambda i: (0, 0, 0)),
            pl.BlockSpec((mid, oc), lambda i: (0, 0)),
            pl.BlockSpec((ic, oc), lambda i: (0, 0)),
            pl.BlockSpec((NEIGH, _SUB * NEIGH), lambda i: (0, 0)),
            pl.BlockSpec((K_PTS * _SUB, _SUB * NEIGH), lambda i: (0, 0)),
        ],
        out_specs=pl.BlockSpec((_NB, oc), lambda i: (i, 0)),
        out_shape=jax.ShapeDtypeStruct((b * n, oc), jnp.float32),
        scratch_shapes=[pltpu.VMEM((K_PTS, _NB, mid), jnp.float32)],
    )(nx, wt, x, wk, wup, wsc, rep, bmask)


def kernel(coords, features, KP, W_down1, Wk1, Wup1, Wsc1, W_down2, Wk2,
           Wup2, Wsc2, W_down3, Wk3, Wup3, Wsc3):
    b, n, _ = coords.shape
    m = b * n * NEIGH

    coords_t = jnp.transpose(coords, (0, 2, 1))
    gidx = _knn(coords, coords_t)
    flat_idx = jnp.transpose(gidx, (0, 2, 1)).reshape(m)

    pts_flat = coords.reshape(b * n, 3)
    pts_pad = jnp.concatenate(
        [pts_flat, jnp.zeros((b * n, 13), jnp.float32)], axis=1
    )
    neigh_c = _sc_gather(pts_pad, flat_idx, tiled=False)
    wt = _wprep(neigh_c, pts_flat, KP, b, n)

    rep = (
        lax.broadcasted_iota(jnp.int32, (NEIGH, _SUB * NEIGH), 1) % NEIGH
        == lax.broadcasted_iota(jnp.int32, (NEIGH, _SUB * NEIGH), 0)
    ).astype(jnp.float32)
    bmask = (
        lax.broadcasted_iota(jnp.int32, (K_PTS * _SUB, _SUB * NEIGH), 0) % _SUB
        == lax.broadcasted_iota(jnp.int32, (K_PTS * _SUB, _SUB * NEIGH), 1)
        // NEIGH
    ).astype(jnp.float32)

    x = features.reshape(b * n, features.shape[2])
    for wd, wk, wup, wsc in (
        (W_down1, Wk1, Wup1, Wsc1),
        (W_down2, Wk2, Wup2, Wsc2),
        (W_down3, Wk3, Wup3, Wsc3),
    ):
        h = _dense_down(x, wd)
        nx = _sc_gather(h, flat_idx)
        x = _conv_block(nx, wt, x, wk, wup, wsc, rep, bmask, b, n)

    return x.reshape(b, n, -1)
